# baseline (speedup 1.0000x reference)
import jax
import jax.numpy as jnp
from jax.experimental import pallas as pl
from jax.experimental.pallas import tpu as pltpu

_BM = 512

_MINV = [
    [-3.0, 27.0, 27.0, -3.0],
    [2.0, -54.0, 54.0, -2.0],
    [12.0, -12.0, -12.0, 12.0],
    [-8.0, 24.0, -24.0, 8.0],
]


def _mlp_kernel(x_ref, k1_ref, k2_ref, k3_ref, b1_ref, w2_ref, b2_ref, out_ref):
    w = x_ref[...].astype(jnp.bfloat16) - jnp.asarray(1.5, jnp.bfloat16)
    w2 = w * w
    w3 = w2 * w
    h = jnp.dot(w, k1_ref[...], preferred_element_type=jnp.float32)
    h = h + jnp.dot(w2, k2_ref[...], preferred_element_type=jnp.float32)
    h = h + jnp.dot(w3, k3_ref[...], preferred_element_type=jnp.float32)
    h = jnp.maximum(h + b1_ref[...], 0.0)
    out_ref[...] = jnp.dot(h, w2_ref[...],
                           preferred_element_type=jnp.float32) + b2_ref[...]


def kernel(x, table, W1, b1, W2, b2):
    B, T = x.shape
    d_hid = W1.shape[1]
    d_out = W2.shape[1]
    A = (jnp.asarray(_MINV, jnp.float32) / 48.0) @ table
    w1p = W1.reshape(T, 2, d_hid)
    we = w1p[:, 0, :]
    wo = w1p[:, 1, :]
    k1 = (A[1, 0] * we + A[1, 1] * wo).astype(jnp.bfloat16)
    k2 = (A[2, 0] * we + A[2, 1] * wo).astype(jnp.bfloat16)
    k3 = (A[3, 0] * we + A[3, 1] * wo).astype(jnp.bfloat16)
    b1p = (b1 + A[0, 0] * jnp.sum(we, axis=0)
           + A[0, 1] * jnp.sum(wo, axis=0)).reshape(1, d_hid)
    b2r = b2.reshape(1, d_out)
    return pl.pallas_call(
        _mlp_kernel,
        grid=(B // _BM,),
        in_specs=[
            pl.BlockSpec((_BM, T), lambda i: (i, 0)),
            pl.BlockSpec((T, d_hid), lambda i: (0, 0)),
            pl.BlockSpec((T, d_hid), lambda i: (0, 0)),
            pl.BlockSpec((T, d_hid), lambda i: (0, 0)),
            pl.BlockSpec((1, d_hid), lambda i: (0, 0)),
            pl.BlockSpec((d_hid, d_out), lambda i: (0, 0)),
            pl.BlockSpec((1, d_out), lambda i: (0, 0)),
        ],
        out_specs=pl.BlockSpec((_BM, d_out), lambda i: (i, 0)),
        out_shape=jax.ShapeDtypeStruct((B, d_out), jnp.float32),
        compiler_params=pltpu.CompilerParams(
            dimension_semantics=("arbitrary",)),
    )(x, k1, k2, k3, b1p, W2, b2r)


# speedup vs baseline: 1.3701x; 1.3701x over previous
import jax
import jax.numpy as jnp
from jax.experimental import pallas as pl
from jax.experimental.pallas import tpu as pltpu

_BM = 512


def _mlp_kernel(coef_ref, x_ref, w1_ref, b1_ref, w2_ref, b2_ref, out_ref):
    xb = x_ref[...]
    v0 = (xb & 1).astype(jnp.bfloat16)
    v1 = (xb >> 1).astype(jnp.bfloat16)
    p = v0 * v1
    c = coef_ref[...].astype(jnp.bfloat16)
    g0 = c[0:1, 0:1] + c[0:1, 1:2] * v0 + c[0:1, 2:3] * v1 + c[0:1, 3:4] * p
    g1 = c[0:1, 4:5] + c[0:1, 5:6] * v0 + c[0:1, 6:7] * v1 + c[0:1, 7:8] * p
    w1 = w1_ref[...]
    n = w1.shape[1] // 2
    h = jnp.dot(g0, w1[:, :n], preferred_element_type=jnp.float32)
    h = h + jnp.dot(g1, w1[:, n:], preferred_element_type=jnp.float32)
    h = jnp.maximum(h + b1_ref[...], 0.0)
    out_ref[...] = jnp.dot(h, w2_ref[...],
                           preferred_element_type=jnp.float32) + b2_ref[...]


def kernel(x, table, W1, b1, W2, b2):
    B, T = x.shape
    d_hid = W1.shape[1]
    d_out = W2.shape[1]
    t = table
    coef = jnp.stack([
        t[0, 0], t[1, 0] - t[0, 0], t[2, 0] - t[0, 0],
        t[3, 0] - t[2, 0] - t[1, 0] + t[0, 0],
        t[0, 1], t[1, 1] - t[0, 1], t[2, 1] - t[0, 1],
        t[3, 1] - t[2, 1] - t[1, 1] + t[0, 1],
    ]).reshape(1, 8)
    w1r = W1.reshape(T, 2 * d_hid).astype(jnp.bfloat16)
    b1r = b1.reshape(1, d_hid)
    b2r = b2.reshape(1, d_out)
    return pl.pallas_call(
        _mlp_kernel,
        grid=(B // _BM,),
        in_specs=[
            pl.BlockSpec((1, 8), lambda i: (0, 0)),
            pl.BlockSpec((_BM, T), lambda i: (i, 0)),
            pl.BlockSpec((T, 2 * d_hid), lambda i: (0, 0)),
            pl.BlockSpec((1, d_hid), lambda i: (0, 0)),
            pl.BlockSpec((d_hid, d_out), lambda i: (0, 0)),
            pl.BlockSpec((1, d_out), lambda i: (0, 0)),
        ],
        out_specs=pl.BlockSpec((_BM, d_out), lambda i: (i, 0)),
        out_shape=jax.ShapeDtypeStruct((B, d_out), jnp.float32),
        compiler_params=pltpu.CompilerParams(
            dimension_semantics=("arbitrary",)),
    )(coef, x, w1r, b1r, W2, b2r)


# speedup vs baseline: 1.3848x; 1.0108x over previous
import jax
import jax.numpy as jnp
from jax.experimental import pallas as pl
from jax.experimental.pallas import tpu as pltpu

_BM = 512


_KC = 1024


def _mlp_kernel(coef_ref, x_ref, w1_ref, b1_ref, w2_ref, b2_ref, out_ref):
    T = x_ref.shape[1]
    n = w1_ref.shape[1] // 2
    c = coef_ref[...].astype(jnp.bfloat16)
    h = None
    for c0 in range(0, T, _KC):
        xb = x_ref[:, c0:c0 + _KC]
        v0 = (xb & 1).astype(jnp.bfloat16)
        v1 = (xb >> 1).astype(jnp.bfloat16)
        p = v0 * v1
        g0 = c[0:1, 0:1] + c[0:1, 1:2] * v0 + c[0:1, 2:3] * v1 + c[0:1, 3:4] * p
        g1 = c[0:1, 4:5] + c[0:1, 5:6] * v0 + c[0:1, 6:7] * v1 + c[0:1, 7:8] * p
        d = jnp.dot(g0, w1_ref[c0:c0 + _KC, :n],
                    preferred_element_type=jnp.float32)
        d = d + jnp.dot(g1, w1_ref[c0:c0 + _KC, n:],
                        preferred_element_type=jnp.float32)
        h = d if h is None else h + d
    h = jnp.maximum(h + b1_ref[...], 0.0)
    out_ref[...] = jnp.dot(h, w2_ref[...],
                           preferred_element_type=jnp.float32) + b2_ref[...]


def kernel(x, table, W1, b1, W2, b2):
    B, T = x.shape
    d_hid = W1.shape[1]
    d_out = W2.shape[1]
    t = table
    coef = jnp.stack([
        t[0, 0], t[1, 0] - t[0, 0], t[2, 0] - t[0, 0],
        t[3, 0] - t[2, 0] - t[1, 0] + t[0, 0],
        t[0, 1], t[1, 1] - t[0, 1], t[2, 1] - t[0, 1],
        t[3, 1] - t[2, 1] - t[1, 1] + t[0, 1],
    ]).reshape(1, 8)
    w1r = W1.reshape(T, 2 * d_hid).astype(jnp.bfloat16)
    b1r = b1.reshape(1, d_hid)
    b2r = b2.reshape(1, d_out)
    return pl.pallas_call(
        _mlp_kernel,
        grid=(B // _BM,),
        in_specs=[
            pl.BlockSpec((1, 8), lambda i: (0, 0)),
            pl.BlockSpec((_BM, T), lambda i: (i, 0)),
            pl.BlockSpec((T, 2 * d_hid), lambda i: (0, 0)),
            pl.BlockSpec((1, d_hid), lambda i: (0, 0)),
            pl.BlockSpec((d_hid, d_out), lambda i: (0, 0)),
            pl.BlockSpec((1, d_out), lambda i: (0, 0)),
        ],
        out_specs=pl.BlockSpec((_BM, d_out), lambda i: (i, 0)),
        out_shape=jax.ShapeDtypeStruct((B, d_out), jnp.float32),
        compiler_params=pltpu.CompilerParams(
            dimension_semantics=("arbitrary",)),
    )(coef, x, w1r, b1r, W2, b2r)
